# baseline (speedup 1.0000x reference)
import jax
import jax.numpy as jnp
from jax import lax
from jax.experimental import pallas as pl
from jax.experimental.pallas import tpu as pltpu

_B, _T, _C = 16, 1024, 999
_N = _B * _T
_ROWS = 512


def _body(x_ref, t_ref, o_ref):
    x = x_ref[...]
    t = t_ref[...]
    col = lax.broadcasted_iota(jnp.int32, (_ROWS, _C), 1)
    hit = col == (t - 1)
    part = jnp.sum(x * x) - 2.0 * jnp.sum(jnp.where(hit, x, 0.0))

    @pl.when(pl.program_id(0) == 0)
    def _():
        o_ref[0, 0] = 0.0

    o_ref[0, 0] += part


def kernel(rel_ress, targets, mask):
    del mask
    x = rel_ress.reshape(_N, _C)
    t_col = targets.astype(jnp.int32).reshape(_N, 1)
    out = pl.pallas_call(
        _body,
        grid=(_N // _ROWS,),
        in_specs=[
            pl.BlockSpec((_ROWS, _C), lambda i: (i, 0)),
            pl.BlockSpec((_ROWS, 1), lambda i: (i, 0)),
        ],
        out_specs=pl.BlockSpec(memory_space=pltpu.SMEM),
        out_shape=jax.ShapeDtypeStruct((1, 1), jnp.float32),
    )(x, t_col)
    return (out[0, 0] + jnp.float32(_N)) / jnp.float32(_N * (_C + 1))
